# Initial kernel scaffold; baseline (speedup 1.0000x reference)
import jax
import jax.numpy as jnp
from jax.experimental import pallas as pl


def kernel(x, edge_index, W, b):
    raise NotImplementedError("write your pallas kernel here")


# speedup vs baseline: 97.0157x; 97.0157x over previous
import functools

import jax
import jax.numpy as jnp
from jax import lax
from jax.experimental import pallas as pl
from jax.experimental.pallas import tpu as pltpu
from jax.experimental.pallas import tpu_sc as plsc

N_NODES = 10000
N_EDGES = 320000
FDIM = 128

NC = 2
NS = 16

NPAD = 10240
ROWS_PER_TILE = 80
ROWS = ROWS_PER_TILE * NC * NS
EPAD = ROWS * 128
ROWS_PER_CORE = ROWS // NC

_MESH = plsc.VectorSubcoreMesh(core_axis_name="c", subcore_axis_name="s")


def _deg_body(dst_hbm, zeros_hbm, out_hbm, idx_v, ones_v, deg_sp):
    cid = lax.axis_index("c")
    sid = lax.axis_index("s")
    base = cid * ROWS_PER_CORE + sid * ROWS_PER_TILE

    for i in range(8):
        ones_v[pl.ds(i * 16, 16)] = jnp.full((16,), 1.0, jnp.float32)

    @pl.when(sid == 0)
    def _():
        pltpu.sync_copy(zeros_hbm, deg_sp)

    pltpu.sync_copy(dst_hbm.at[pl.ds(base, ROWS_PER_TILE)], idx_v)
    plsc.subcore_barrier()

    def body(j, carry):
        pltpu.sync_copy(ones_v, deg_sp.at[idx_v.at[j]], add=True)
        return carry

    lax.fori_loop(0, ROWS_PER_TILE, body, 0)
    plsc.subcore_barrier()

    @pl.when(sid == 0)
    def _():
        pltpu.sync_copy(deg_sp, out_hbm.at[cid])


_deg_kernel = functools.partial(
    pl.kernel,
    out_type=jax.ShapeDtypeStruct((NC, NPAD), jnp.float32),
    mesh=_MESH,
    scratch_types=[
        pltpu.VMEM((ROWS_PER_TILE, 128), jnp.int32),
        pltpu.VMEM((128,), jnp.float32),
        pltpu.VMEM_SHARED((NPAD,), jnp.float32),
    ],
)(_deg_body)


def _s_body(src_hbm, dst_hbm, dinv_hbm, zeros_hbm, out_hbm,
            sidx_v, didx_v, vals_v, dinv_sp, s_sp):
    cid = lax.axis_index("c")
    sid = lax.axis_index("s")
    base = cid * ROWS_PER_CORE + sid * ROWS_PER_TILE

    @pl.when(sid == 0)
    def _():
        pltpu.sync_copy(zeros_hbm, s_sp)
        pltpu.sync_copy(dinv_hbm, dinv_sp)

    pltpu.sync_copy(src_hbm.at[pl.ds(base, ROWS_PER_TILE)], sidx_v)
    pltpu.sync_copy(dst_hbm.at[pl.ds(base, ROWS_PER_TILE)], didx_v)
    plsc.subcore_barrier()

    def body(j, carry):
        pltpu.sync_copy(dinv_sp.at[didx_v.at[j]], vals_v)
        pltpu.sync_copy(vals_v, s_sp.at[sidx_v.at[j]], add=True)
        return carry

    lax.fori_loop(0, ROWS_PER_TILE, body, 0)
    plsc.subcore_barrier()

    @pl.when(sid == 0)
    def _():
        pltpu.sync_copy(s_sp, out_hbm.at[cid])


_s_kernel = functools.partial(
    pl.kernel,
    out_type=jax.ShapeDtypeStruct((NC, NPAD), jnp.float32),
    mesh=_MESH,
    scratch_types=[
        pltpu.VMEM((ROWS_PER_TILE, 128), jnp.int32),
        pltpu.VMEM((ROWS_PER_TILE, 128), jnp.int32),
        pltpu.VMEM((128,), jnp.float32),
        pltpu.VMEM_SHARED((NPAD,), jnp.float32),
        pltpu.VMEM_SHARED((NPAD,), jnp.float32),
    ],
)(_s_body)


def _dinv_body(d0_ref, d1_ref, out_ref):
    deg = d0_ref[...] + d1_ref[...] + 1.0
    r = lax.rsqrt(deg)
    row = lax.broadcasted_iota(jnp.int32, (NPAD // 128, 128), 0)
    col = lax.broadcasted_iota(jnp.int32, (NPAD // 128, 128), 1)
    idx = row * 128 + col
    out_ref[...] = jnp.where(idx < N_NODES, r, 0.0)


def _dinv_call(d0, d1):
    return pl.pallas_call(
        _dinv_body,
        out_shape=jax.ShapeDtypeStruct((NPAD // 128, 128), jnp.float32),
    )(d0, d1)


def _out_body(dinv_ref, s0_ref, s1_ref, x_ref, wt_ref, b_ref, out_ref):
    dinv = dinv_ref[...]
    coef = dinv * (s0_ref[...] + s1_ref[...] + dinv)
    acc = jnp.dot(coef, x_ref[...], preferred_element_type=jnp.float32)
    out_ref[...] = (
        jnp.dot(acc, wt_ref[...], preferred_element_type=jnp.float32)
        + float(N_NODES) * b_ref[...]
    )


def _out_call(dinv, s0, s1, x_pad, wt, b2):
    return pl.pallas_call(
        _out_body,
        out_shape=jax.ShapeDtypeStruct((1, FDIM), jnp.float32),
    )(dinv, s0, s1, x_pad, wt, b2)


@jax.jit
def kernel(x, edge_index, W, b):
    pad = N_NODES + 200 + (jnp.arange(EPAD - N_EDGES, dtype=jnp.int32) % 32)
    srcp = jnp.concatenate([edge_index[0], pad]).reshape(ROWS, 128)
    dstp = jnp.concatenate([edge_index[1], pad]).reshape(ROWS, 128)
    zeros = jnp.zeros((NPAD,), jnp.float32)

    deg_part = _deg_kernel(dstp, zeros)
    dinv2 = _dinv_call(deg_part[0].reshape(NPAD // 128, 128),
                       deg_part[1].reshape(NPAD // 128, 128))
    dinv_flat = dinv2.reshape(NPAD)
    s_part = _s_kernel(srcp, dstp, dinv_flat, zeros)

    x_pad = jnp.pad(x, ((0, NPAD - N_NODES), (0, 0)))
    out = _out_call(
        dinv_flat.reshape(1, NPAD),
        s_part[0].reshape(1, NPAD),
        s_part[1].reshape(1, NPAD),
        x_pad,
        W.T,
        b.reshape(1, FDIM),
    )
    return out


# speedup vs baseline: 114.0682x; 1.1758x over previous
import functools

import jax
import jax.numpy as jnp
from jax import lax
from jax.experimental import pallas as pl
from jax.experimental.pallas import tpu as pltpu
from jax.experimental.pallas import tpu_sc as plsc

N_NODES = 10000
N_EDGES = 320000
FDIM = 128

NC = 2
NS = 16

NPAD = 10240
EDGES_PER_TILE = 10240
EPAD = EDGES_PER_TILE * NC * NS

_MESH = plsc.VectorSubcoreMesh(core_axis_name="c", subcore_axis_name="s")


def _deg_body(dst_hbm, ones_hbm, zeros_hbm, out_hbm, idx_v, ones_v, deg_sp):
    cid = lax.axis_index("c")
    sid = lax.axis_index("s")
    base = (cid * NS + sid) * EDGES_PER_TILE

    pltpu.sync_copy(ones_hbm, ones_v)

    @pl.when(sid == 0)
    def _():
        pltpu.sync_copy(zeros_hbm, deg_sp)

    pltpu.sync_copy(dst_hbm.at[pl.ds(base, EDGES_PER_TILE)], idx_v)
    plsc.subcore_barrier()

    pltpu.sync_copy(ones_v, deg_sp.at[idx_v], add=True)
    plsc.subcore_barrier()

    @pl.when(sid == 0)
    def _():
        pltpu.sync_copy(deg_sp, out_hbm.at[cid])


_deg_kernel = functools.partial(
    pl.kernel,
    out_type=jax.ShapeDtypeStruct((NC, NPAD), jnp.float32),
    mesh=_MESH,
    scratch_types=[
        pltpu.VMEM((EDGES_PER_TILE,), jnp.int32),
        pltpu.VMEM((EDGES_PER_TILE,), jnp.float32),
        pltpu.VMEM_SHARED((NPAD,), jnp.float32),
    ],
)(_deg_body)


def _s_body(src_hbm, dst_hbm, dinv_hbm, zeros_hbm, out_hbm,
            sidx_v, didx_v, vals_v, dinv_sp, s_sp):
    cid = lax.axis_index("c")
    sid = lax.axis_index("s")
    base = (cid * NS + sid) * EDGES_PER_TILE

    @pl.when(sid == 0)
    def _():
        pltpu.sync_copy(zeros_hbm, s_sp)
        pltpu.sync_copy(dinv_hbm, dinv_sp)

    pltpu.sync_copy(src_hbm.at[pl.ds(base, EDGES_PER_TILE)], sidx_v)
    pltpu.sync_copy(dst_hbm.at[pl.ds(base, EDGES_PER_TILE)], didx_v)
    plsc.subcore_barrier()

    pltpu.sync_copy(dinv_sp.at[didx_v], vals_v)
    pltpu.sync_copy(vals_v, s_sp.at[sidx_v], add=True)
    plsc.subcore_barrier()

    @pl.when(sid == 0)
    def _():
        pltpu.sync_copy(s_sp, out_hbm.at[cid])


_s_kernel = functools.partial(
    pl.kernel,
    out_type=jax.ShapeDtypeStruct((NC, NPAD), jnp.float32),
    mesh=_MESH,
    scratch_types=[
        pltpu.VMEM((EDGES_PER_TILE,), jnp.int32),
        pltpu.VMEM((EDGES_PER_TILE,), jnp.int32),
        pltpu.VMEM((EDGES_PER_TILE,), jnp.float32),
        pltpu.VMEM_SHARED((NPAD,), jnp.float32),
        pltpu.VMEM_SHARED((NPAD,), jnp.float32),
    ],
)(_s_body)


def _dinv_body(d0_ref, d1_ref, out_ref):
    deg = d0_ref[...] + d1_ref[...] + 1.0
    r = lax.rsqrt(deg)
    row = lax.broadcasted_iota(jnp.int32, (NPAD // 128, 128), 0)
    col = lax.broadcasted_iota(jnp.int32, (NPAD // 128, 128), 1)
    idx = row * 128 + col
    out_ref[...] = jnp.where(idx < N_NODES, r, 0.0)


def _dinv_call(d0, d1):
    return pl.pallas_call(
        _dinv_body,
        out_shape=jax.ShapeDtypeStruct((NPAD // 128, 128), jnp.float32),
    )(d0, d1)


def _out_body(dinv_ref, s0_ref, s1_ref, x_ref, wt_ref, b_ref, out_ref):
    dinv = dinv_ref[...]
    coef = dinv * (s0_ref[...] + s1_ref[...] + dinv)
    acc = jnp.dot(coef, x_ref[...], preferred_element_type=jnp.float32)
    out_ref[...] = (
        jnp.dot(acc, wt_ref[...], preferred_element_type=jnp.float32)
        + float(N_NODES) * b_ref[...]
    )


def _out_call(dinv, s0, s1, x_pad, wt, b2):
    return pl.pallas_call(
        _out_body,
        out_shape=jax.ShapeDtypeStruct((1, FDIM), jnp.float32),
    )(dinv, s0, s1, x_pad, wt, b2)


@jax.jit
def kernel(x, edge_index, W, b):
    pad = N_NODES + 200 + (jnp.arange(EPAD - N_EDGES, dtype=jnp.int32) % 32)
    srcp = jnp.concatenate([edge_index[0], pad])
    dstp = jnp.concatenate([edge_index[1], pad])
    zeros = jnp.zeros((NPAD,), jnp.float32)
    ones = jnp.ones((EDGES_PER_TILE,), jnp.float32)

    deg_part = _deg_kernel(dstp, ones, zeros)
    dinv2 = _dinv_call(deg_part[0].reshape(NPAD // 128, 128),
                       deg_part[1].reshape(NPAD // 128, 128))
    dinv_flat = dinv2.reshape(NPAD)
    s_part = _s_kernel(srcp, dstp, dinv_flat, zeros)

    x_pad = jnp.pad(x, ((0, NPAD - N_NODES), (0, 0)))
    out = _out_call(
        dinv_flat.reshape(1, NPAD),
        s_part[0].reshape(1, NPAD),
        s_part[1].reshape(1, NPAD),
        x_pad,
        W.T,
        b.reshape(1, FDIM),
    )
    return out


# speedup vs baseline: 167.3236x; 1.4669x over previous
import functools

import jax
import jax.numpy as jnp
from jax import lax
from jax.experimental import pallas as pl
from jax.experimental.pallas import tpu as pltpu
from jax.experimental.pallas import tpu_sc as plsc

N_NODES = 10000
N_EDGES = 320000
FDIM = 128

NC = 2
NS = 16
NW = NC * NS

NPAD = 10240
CHUNK = 10240
LAST = N_EDGES - (NW - 1) * CHUNK

_MESH = plsc.VectorSubcoreMesh(core_axis_name="c", subcore_axis_name="s")


def _deg_body(edge_hbm, ones_hbm, zeros_hbm, out0_hbm, out1_hbm,
              dst_v, ones_v, deg_sp):
    cid = lax.axis_index("c")
    sid = lax.axis_index("s")
    wid = cid * NS + sid
    base = pl.multiple_of(wid * CHUNK, 128)

    pltpu.sync_copy(ones_hbm, ones_v)

    @pl.when(sid == 0)
    def _():
        pltpu.sync_copy(zeros_hbm.at[0], deg_sp)

    @pl.when(wid < NW - 1)
    def _():
        pltpu.sync_copy(edge_hbm.at[1, pl.ds(base, CHUNK)], dst_v)

    @pl.when(wid == NW - 1)
    def _():
        pltpu.sync_copy(edge_hbm.at[1, pl.ds(base, LAST)],
                        dst_v.at[pl.ds(0, LAST)])

    plsc.subcore_barrier()

    @pl.when(wid < NW - 1)
    def _():
        pltpu.sync_copy(ones_v, deg_sp.at[dst_v], add=True)

    @pl.when(wid == NW - 1)
    def _():
        pltpu.sync_copy(ones_v.at[pl.ds(0, LAST)],
                        deg_sp.at[dst_v.at[pl.ds(0, LAST)]], add=True)

    plsc.subcore_barrier()

    @pl.when(sid == 0)
    def _():
        @pl.when(cid == 0)
        def _():
            pltpu.sync_copy(deg_sp, out0_hbm.at[0])

        @pl.when(cid == 1)
        def _():
            pltpu.sync_copy(deg_sp, out1_hbm.at[0])


_deg_kernel = functools.partial(
    pl.kernel,
    out_type=(jax.ShapeDtypeStruct((1, NPAD), jnp.float32),
              jax.ShapeDtypeStruct((1, NPAD), jnp.float32)),
    mesh=_MESH,
    scratch_types=[
        pltpu.VMEM((CHUNK,), jnp.int32),
        pltpu.VMEM((CHUNK,), jnp.float32),
        pltpu.VMEM_SHARED((NPAD,), jnp.float32),
    ],
)(_deg_body)


def _dinv_body(d0_ref, d1_ref, out_ref):
    deg = d0_ref[...] + d1_ref[...] + 1.0
    r = lax.rsqrt(deg)
    idx = lax.broadcasted_iota(jnp.int32, (1, NPAD), 1)
    out_ref[...] = jnp.where(idx < N_NODES, r, 0.0)


def _dinv_call(d0, d1):
    return pl.pallas_call(
        _dinv_body,
        out_shape=jax.ShapeDtypeStruct((1, NPAD), jnp.float32),
    )(d0, d1)


def _s_body(edge_hbm, dinv_hbm, zeros_hbm, s0_hbm, s1_hbm,
            src_v, dst_v, vals_v, dinv_sp, s_sp):
    cid = lax.axis_index("c")
    sid = lax.axis_index("s")
    wid = cid * NS + sid
    base = pl.multiple_of(wid * CHUNK, 128)

    @pl.when(sid == 0)
    def _():
        pltpu.sync_copy(zeros_hbm.at[0], s_sp)
        pltpu.sync_copy(dinv_hbm.at[0], dinv_sp)

    @pl.when(wid < NW - 1)
    def _():
        pltpu.sync_copy(edge_hbm.at[0, pl.ds(base, CHUNK)], src_v)
        pltpu.sync_copy(edge_hbm.at[1, pl.ds(base, CHUNK)], dst_v)

    @pl.when(wid == NW - 1)
    def _():
        pltpu.sync_copy(edge_hbm.at[0, pl.ds(base, LAST)],
                        src_v.at[pl.ds(0, LAST)])
        pltpu.sync_copy(edge_hbm.at[1, pl.ds(base, LAST)],
                        dst_v.at[pl.ds(0, LAST)])

    plsc.subcore_barrier()

    @pl.when(wid < NW - 1)
    def _():
        pltpu.sync_copy(dinv_sp.at[dst_v], vals_v)
        pltpu.sync_copy(vals_v, s_sp.at[src_v], add=True)

    @pl.when(wid == NW - 1)
    def _():
        pltpu.sync_copy(dinv_sp.at[dst_v.at[pl.ds(0, LAST)]],
                        vals_v.at[pl.ds(0, LAST)])
        pltpu.sync_copy(vals_v.at[pl.ds(0, LAST)],
                        s_sp.at[src_v.at[pl.ds(0, LAST)]], add=True)

    plsc.subcore_barrier()

    @pl.when(sid == 0)
    def _():
        @pl.when(cid == 0)
        def _():
            pltpu.sync_copy(s_sp, s0_hbm.at[0])

        @pl.when(cid == 1)
        def _():
            pltpu.sync_copy(s_sp, s1_hbm.at[0])


_s_kernel = functools.partial(
    pl.kernel,
    out_type=(jax.ShapeDtypeStruct((1, NPAD), jnp.float32),
              jax.ShapeDtypeStruct((1, NPAD), jnp.float32)),
    mesh=_MESH,
    scratch_types=[
        pltpu.VMEM((CHUNK,), jnp.int32),
        pltpu.VMEM((CHUNK,), jnp.int32),
        pltpu.VMEM((CHUNK,), jnp.float32),
        pltpu.VMEM_SHARED((NPAD,), jnp.float32),
        pltpu.VMEM_SHARED((NPAD,), jnp.float32),
    ],
)(_s_body)


def _out_body(dinv_ref, s0_ref, s1_ref, x_ref, wt_ref, b_ref, out_ref):
    dinv = dinv_ref[...]
    coef = dinv * (s0_ref[...] + s1_ref[...] + dinv)
    acc = jnp.dot(coef, x_ref[...], preferred_element_type=jnp.float32)
    out_ref[...] = (
        jnp.dot(acc, wt_ref[...], preferred_element_type=jnp.float32)
        + float(N_NODES) * b_ref[...]
    )


def _out_call(dinv, s0, s1, x_pad, wt, b2):
    return pl.pallas_call(
        _out_body,
        out_shape=jax.ShapeDtypeStruct((1, FDIM), jnp.float32),
    )(dinv, s0, s1, x_pad, wt, b2)


@jax.jit
def kernel(x, edge_index, W, b):
    zeros = jnp.zeros((1, NPAD), jnp.float32)
    ones = jnp.ones((CHUNK,), jnp.float32)

    deg0, deg1 = _deg_kernel(edge_index, ones, zeros)
    dinv = _dinv_call(deg0, deg1)
    s0, s1 = _s_kernel(edge_index, dinv, zeros)

    x_pad = jnp.pad(x, ((0, NPAD - N_NODES), (0, 0)))
    return _out_call(dinv, s0, s1, x_pad, W.T, b.reshape(1, FDIM))


# speedup vs baseline: 167.6297x; 1.0018x over previous
import functools

import jax
import jax.numpy as jnp
import numpy as np
from jax import lax
from jax.experimental import pallas as pl
from jax.experimental.pallas import tpu as pltpu
from jax.experimental.pallas import tpu_sc as plsc

N_NODES = 10000
N_EDGES = 320000
FDIM = 128

NC = 2
NS = 16
NW = NC * NS

NPAD = 10240
SLICE = NPAD // NS
CHUNK = 10240
LAST = N_EDGES - (NW - 1) * CHUNK
NSPLIT = 9984

_MESH = plsc.VectorSubcoreMesh(core_axis_name="c", subcore_axis_name="s")

_ONES = np.ones((CHUNK,), np.float32)


def _rsqrt_vec(d):
    m = d
    scale = jnp.full((16,), 1.0, jnp.float32)
    for _ in range(9):
        big = m > 4.0
        m = jnp.where(big, m * 0.25, m)
        scale = jnp.where(big, scale * 0.5, scale)
    y = scale * 0.7
    half_d = 0.5 * d
    for _ in range(5):
        y = y * (1.5 - half_d * y * y)
    return y


def _deg_body(edge_hbm, ones_hbm, out0_hbm, out1_hbm,
              dst_v, ones_v, zslice_v, deg_sp):
    cid = lax.axis_index("c")
    sid = lax.axis_index("s")
    wid = cid * NS + sid
    base = pl.multiple_of(wid * CHUNK, 128)
    nbase = sid * SLICE

    pltpu.sync_copy(ones_hbm, ones_v)

    def zbody(j, carry):
        zslice_v[pl.ds(j * 16, 16)] = jnp.full((16,), 0.0, jnp.float32)
        return carry

    lax.fori_loop(0, SLICE // 16, zbody, 0)
    pltpu.sync_copy(zslice_v, deg_sp.at[pl.ds(nbase, SLICE)])

    @pl.when(wid < NW - 1)
    def _():
        pltpu.sync_copy(edge_hbm.at[1, pl.ds(base, CHUNK)], dst_v)

    @pl.when(wid == NW - 1)
    def _():
        pltpu.sync_copy(edge_hbm.at[1, pl.ds(base, LAST)],
                        dst_v.at[pl.ds(0, LAST)])

    plsc.subcore_barrier()

    @pl.when(wid < NW - 1)
    def _():
        pltpu.sync_copy(ones_v, deg_sp.at[dst_v], add=True)

    @pl.when(wid == NW - 1)
    def _():
        pltpu.sync_copy(ones_v.at[pl.ds(0, LAST)],
                        deg_sp.at[dst_v.at[pl.ds(0, LAST)]], add=True)

    plsc.subcore_barrier()

    @pl.when(sid == 0)
    def _():
        @pl.when(cid == 0)
        def _():
            pltpu.sync_copy(deg_sp, out0_hbm.at[0])

        @pl.when(cid == 1)
        def _():
            pltpu.sync_copy(deg_sp, out1_hbm.at[0])


_deg_kernel = functools.partial(
    pl.kernel,
    out_type=(jax.ShapeDtypeStruct((1, NPAD), jnp.float32),
              jax.ShapeDtypeStruct((1, NPAD), jnp.float32)),
    mesh=_MESH,
    scratch_types=[
        pltpu.VMEM((CHUNK,), jnp.int32),
        pltpu.VMEM((CHUNK,), jnp.float32),
        pltpu.VMEM((SLICE,), jnp.float32),
        pltpu.VMEM_SHARED((NPAD,), jnp.float32),
    ],
)(_deg_body)


def _s_body(edge_hbm, deg0_hbm, deg1_hbm, s0_hbm, s1_hbm,
            src_v, dst_v, vals_v, dbuf, dtile_v, dinv_sp, s_sp):
    cid = lax.axis_index("c")
    sid = lax.axis_index("s")
    wid = cid * NS + sid
    base = pl.multiple_of(wid * CHUNK, 128)
    nbase = sid * SLICE

    @pl.when(wid < NW - 1)
    def _():
        pltpu.sync_copy(edge_hbm.at[0, pl.ds(base, CHUNK)], src_v)
        pltpu.sync_copy(edge_hbm.at[1, pl.ds(base, CHUNK)], dst_v)

    @pl.when(wid == NW - 1)
    def _():
        pltpu.sync_copy(edge_hbm.at[0, pl.ds(base, LAST)],
                        src_v.at[pl.ds(0, LAST)])
        pltpu.sync_copy(edge_hbm.at[1, pl.ds(base, LAST)],
                        dst_v.at[pl.ds(0, LAST)])

    pltpu.sync_copy(deg0_hbm.at[0, pl.ds(nbase, SLICE)], dbuf.at[0])
    pltpu.sync_copy(deg1_hbm.at[0, pl.ds(nbase, SLICE)], dbuf.at[1])

    def rs_body(j, carry):
        off = j * 16
        d = dbuf[0, pl.ds(off, 16)] + dbuf[1, pl.ds(off, 16)] + 1.0
        dtile_v[pl.ds(off, 16)] = _rsqrt_vec(d)
        return carry

    lax.fori_loop(0, SLICE // 16, rs_body, 0)
    pltpu.sync_copy(dtile_v, dinv_sp.at[pl.ds(nbase, SLICE)])

    def zbody(j, carry):
        dtile_v[pl.ds(j * 16, 16)] = jnp.full((16,), 0.0, jnp.float32)
        return carry

    lax.fori_loop(0, SLICE // 16, zbody, 0)
    pltpu.sync_copy(dtile_v, s_sp.at[pl.ds(nbase, SLICE)])

    plsc.subcore_barrier()

    @pl.when(wid < NW - 1)
    def _():
        pltpu.sync_copy(dinv_sp.at[dst_v], vals_v)
        pltpu.sync_copy(vals_v, s_sp.at[src_v], add=True)

    @pl.when(wid == NW - 1)
    def _():
        pltpu.sync_copy(dinv_sp.at[dst_v.at[pl.ds(0, LAST)]],
                        vals_v.at[pl.ds(0, LAST)])
        pltpu.sync_copy(vals_v.at[pl.ds(0, LAST)],
                        s_sp.at[src_v.at[pl.ds(0, LAST)]], add=True)

    plsc.subcore_barrier()

    @pl.when(sid == 0)
    def _():
        @pl.when(cid == 0)
        def _():
            pltpu.sync_copy(s_sp, s0_hbm.at[0])

        @pl.when(cid == 1)
        def _():
            pltpu.sync_copy(s_sp, s1_hbm.at[0])


_s_kernel = functools.partial(
    pl.kernel,
    out_type=(jax.ShapeDtypeStruct((1, NPAD), jnp.float32),
              jax.ShapeDtypeStruct((1, NPAD), jnp.float32)),
    mesh=_MESH,
    scratch_types=[
        pltpu.VMEM((CHUNK,), jnp.int32),
        pltpu.VMEM((CHUNK,), jnp.int32),
        pltpu.VMEM((CHUNK,), jnp.float32),
        pltpu.VMEM((2, SLICE), jnp.float32),
        pltpu.VMEM((SLICE,), jnp.float32),
        pltpu.VMEM_SHARED((NPAD,), jnp.float32),
        pltpu.VMEM_SHARED((NPAD,), jnp.float32),
    ],
)(_s_body)


def _out_body(d0_ref, d1_ref, s0_ref, s1_ref, x_ref, w_ref, b_ref, out_ref):
    deg = d0_ref[...] + d1_ref[...] + 1.0
    idx = lax.broadcasted_iota(jnp.int32, (1, NPAD), 1)
    dinv = jnp.where(idx < N_NODES, lax.rsqrt(deg), 0.0)
    coef = dinv * (s0_ref[...] + s1_ref[...] + dinv)
    acc = jnp.dot(coef[:, :NSPLIT], x_ref[:NSPLIT, :],
                  preferred_element_type=jnp.float32)
    acc = acc + jnp.dot(coef[:, NSPLIT:N_NODES], x_ref[NSPLIT:, :],
                        preferred_element_type=jnp.float32)
    out_ref[...] = (
        lax.dot_general(acc, w_ref[...], (((1,), (1,)), ((), ())),
                        preferred_element_type=jnp.float32)
        + float(N_NODES) * b_ref[...]
    )


def _out_call(d0, d1, s0, s1, x, w, b2):
    return pl.pallas_call(
        _out_body,
        out_shape=jax.ShapeDtypeStruct((1, FDIM), jnp.float32),
    )(d0, d1, s0, s1, x, w, b2)


@jax.jit
def kernel(x, edge_index, W, b):
    ones = jnp.asarray(_ONES)
    deg0, deg1 = _deg_kernel(edge_index, ones)
    s0, s1 = _s_kernel(edge_index, deg0, deg1)
    return _out_call(deg0, deg1, s0, s1, x, W, b.reshape(1, FDIM))
